# baseline (speedup 1.0000x reference)
import functools

import jax
import jax.numpy as jnp
from jax import lax
from jax.experimental import pallas as pl
from jax.experimental.pallas import tpu as pltpu
from jax.experimental.pallas import tpu_sc as plsc

_BUF_ROWS = 96


def _chunks(total, size):
    offs, sizes, o = [], [], 0
    while o < total:
        c = min(size, total - o)
        offs.append(o)
        sizes.append(c)
        o += c
    return offs, sizes


def kernel(x, weight):
    batch, seq = x.shape
    dim = weight.shape[1]
    info = plsc.get_sparse_core_info()
    nw = info.num_cores * info.num_subcores
    rows_per_w = seq // nw
    offs, sizes = _chunks(rows_per_w, _BUF_ROWS)

    mesh = plsc.VectorSubcoreMesh(core_axis_name="c", subcore_axis_name="s")

    @functools.partial(
        pl.kernel,
        mesh=mesh,
        out_type=jax.ShapeDtypeStruct((batch, seq, dim), weight.dtype),
        scratch_types=[
            pltpu.VMEM((_BUF_ROWS, dim), weight.dtype),
            pltpu.SemaphoreType.DMA,
        ],
        compiler_params=pltpu.CompilerParams(use_tc_tiling_on_sc=True),
    )
    def _sc_bcast(w_hbm, o_hbm, buf, sem):
        wid = lax.axis_index("s") * info.num_cores + lax.axis_index("c")
        base = wid * rows_per_w

        for off, size in zip(offs, sizes):
            r0 = base + off
            pltpu.sync_copy(w_hbm.at[pl.ds(r0, size), :], buf.at[pl.ds(0, size)])
            for b in range(batch):
                pltpu.sync_copy(
                    buf.at[pl.ds(0, size)], o_hbm.at[b, pl.ds(r0, size), :]
                )

    return _sc_bcast(weight)


# speedup vs baseline: 1.0037x; 1.0037x over previous
import functools

import jax
import jax.numpy as jnp
from jax import lax
from jax.experimental import pallas as pl
from jax.experimental.pallas import tpu as pltpu
from jax.experimental.pallas import tpu_sc as plsc

_BUF_ROWS = 96


def _chunks(total, size):
    offs, sizes, o = [], [], 0
    while o < total:
        c = min(size, total - o)
        offs.append(o)
        sizes.append(c)
        o += c
    return offs, sizes


def kernel(x, weight):
    batch, seq = x.shape
    dim = weight.shape[1]
    info = plsc.get_sparse_core_info()
    nw = info.num_cores * info.num_subcores
    rows_per_w = seq // nw
    offs, sizes = _chunks(rows_per_w, _BUF_ROWS)

    mesh = plsc.VectorSubcoreMesh(core_axis_name="c", subcore_axis_name="s")

    @functools.partial(
        pl.kernel,
        mesh=mesh,
        out_type=jax.ShapeDtypeStruct((batch, seq, dim), weight.dtype),
        scratch_types=[
            pltpu.VMEM((_BUF_ROWS, dim), weight.dtype),
            pltpu.SemaphoreType.DMA,
        ],
    )
    def _sc_bcast(w_hbm, o_hbm, buf, sem):
        wid = lax.axis_index("s") * info.num_cores + lax.axis_index("c")
        base = wid * rows_per_w

        for off, size in zip(offs, sizes):
            r0 = base + off
            pltpu.sync_copy(w_hbm.at[pl.ds(r0, size), :], buf.at[pl.ds(0, size)])
            for b in range(batch):
                pltpu.sync_copy(
                    buf.at[pl.ds(0, size)], o_hbm.at[b, pl.ds(r0, size), :]
                )

    return _sc_bcast(weight)


# speedup vs baseline: 1.0061x; 1.0024x over previous
import functools

import jax
from jax import lax
from jax.experimental import pallas as pl
from jax.experimental.pallas import tpu as pltpu
from jax.experimental.pallas import tpu_sc as plsc

_BUF_ROWS = 96


def _chunks(total, size):
    offs, sizes, o = [], [], 0
    while o < total:
        c = min(size, total - o)
        offs.append(o)
        sizes.append(c)
        o += c
    return offs, sizes


def kernel(x, weight):
    batch, seq = x.shape
    dim = weight.shape[1]
    info = plsc.get_sparse_core_info()
    nw = info.num_cores * info.num_subcores
    rows_per_w = seq // nw
    offs, sizes = _chunks(rows_per_w, _BUF_ROWS)

    mesh = plsc.VectorSubcoreMesh(core_axis_name="c", subcore_axis_name="s")

    @functools.partial(
        pl.kernel,
        mesh=mesh,
        out_type=jax.ShapeDtypeStruct((batch, seq, dim), weight.dtype),
        scratch_types=[
            pltpu.VMEM((_BUF_ROWS, dim), weight.dtype),
        ],
    )
    def _sc_bcast(w_hbm, o_hbm, buf):
        wid = lax.axis_index("s") * info.num_cores + lax.axis_index("c")
        base = wid * rows_per_w

        for off, size in zip(offs, sizes):
            r0 = base + off
            pltpu.sync_copy(w_hbm.at[pl.ds(r0, size), :], buf.at[pl.ds(0, size)])
            for b in range(batch):
                pltpu.sync_copy(
                    buf.at[pl.ds(0, size)], o_hbm.at[b, pl.ds(r0, size), :]
                )

    return _sc_bcast(weight)
